# Initial kernel scaffold; baseline (speedup 1.0000x reference)
import jax
import jax.numpy as jnp
from jax.experimental import pallas as pl


def kernel(x, plane0, plane1, plane2, plane3):
    raise NotImplementedError("write your pallas kernel here")


# speedup vs baseline: 106.8855x; 106.8855x over previous
import functools

import jax
import jax.numpy as jnp
from jax import lax
from jax.experimental import pallas as pl
from jax.experimental.pallas import tpu as pltpu
from jax.experimental.pallas import tpu_sc as plsc

RES_LIST = (64, 128, 256, 512)
NLVL = 4
NPLANE = 3
DF = 8
NPTS = 1048576
NCORES = 2
NSUB = 16
LANES = 16
NWORK = NCORES * NSUB
PPW = NPTS // NWORK
CH = 128
NCHUNK = PPW // CH
NGRP = CH // LANES
PLANE_DXY = ((0, 2), (1, 0), (2, 1))


def _tec_body(x0, x1, x2, tbl0, tbl1, tbl2, tbl3, out, xs_v, idx_v, w_v, tap_v, out_v, sem):
    tbls = (tbl0, tbl1, tbl2, tbl3)
    xcoords = (x0, x1, x2)
    wid = lax.axis_index("s") * NCORES + lax.axis_index("c")
    base0 = wid * PPW
    iota = lax.iota(jnp.int32, LANES)

    def chunk_body(ci, carry):
        base = base0 + ci * CH

        for d in range(3):
            pltpu.sync_copy(xcoords[d].at[pl.ds(base, CH)], xs_v.at[pl.ds(d * CH, CH)])

        def grp_idx(g, c2):
            g0 = g * LANES
            for l in range(NLVL):
                r = RES_LIST[l]
                st = []
                for d in range(3):
                    xd = xs_v[pl.ds(d * CH + g0, LANES)]
                    t = xd * float(r) - 0.5
                    i = (t + 1.0).astype(jnp.int32) - 1
                    w1 = t - i.astype(jnp.float32)
                    w0 = 1.0 - w1
                    lo = i < 0
                    hi = i > r - 2
                    ib = jnp.minimum(jnp.maximum(i, 0), r - 2)
                    wA = jnp.where(lo, w1, jnp.where(hi, 0.0, w0))
                    wB = jnp.where(hi, w0, jnp.where(lo, 0.0, w1))
                    yc0 = jnp.maximum(i, 0)
                    yc1 = jnp.minimum(i + 1, r - 1)
                    wy0 = jnp.where(lo, 0.0, w0)
                    wy1 = jnp.where(hi, 0.0, w1)
                    st.append((ib, wA, wB, yc0, yc1, wy0, wy1))
                for s in range(NPLANE):
                    dx, dy = PLANE_DXY[s]
                    ibx, wA, wB = st[dx][0], st[dx][1], st[dx][2]
                    yc0, yc1, wy0, wy1 = st[dy][3], st[dy][4], st[dy][5], st[dy][6]
                    lp = l * NPLANE + s
                    srr = s * r * r
                    idx_v[lp, 0, pl.ds(g0, LANES)] = yc0 * r + (ibx + srr)
                    idx_v[lp, 1, pl.ds(g0, LANES)] = yc1 * r + (ibx + srr)
                    w_v[lp, 0, pl.ds(g0, LANES)] = wy0 * wA
                    w_v[lp, 1, pl.ds(g0, LANES)] = wy0 * wB
                    w_v[lp, 2, pl.ds(g0, LANES)] = wy1 * wA
                    w_v[lp, 3, pl.ds(g0, LANES)] = wy1 * wB
            return c2

        lax.fori_loop(0, NGRP, grp_idx, 0)

        copies = []
        for l in range(NLVL):
            for s in range(NPLANE):
                lp = l * NPLANE + s
                for j in range(2):
                    copies.append(pltpu.async_copy(
                        tbls[l].at[idx_v.at[lp, j]],
                        tap_v.at[pl.ds((lp * 2 + j) * CH, CH), :],
                        sem))
        for cp in copies:
            cp.wait()

        def grp_comb(g, c2):
            g0 = g * LANES
            pt = g0 + iota
            pt96 = pt * (NLVL * NPLANE * DF)
            for lp in range(12):
                w00 = w_v[lp, 0, pl.ds(g0, LANES)]
                w01 = w_v[lp, 1, pl.ds(g0, LANES)]
                w10 = w_v[lp, 2, pl.ds(g0, LANES)]
                w11 = w_v[lp, 3, pl.ds(g0, LANES)]
                row0 = pt + (lp * 2) * CH
                row1 = pt + (lp * 2 + 1) * CH
                l, s = lp // NPLANE, lp % NPLANE
                for c in range(DF):
                    cA = jnp.full((LANES,), c, jnp.int32)
                    cB = jnp.full((LANES,), c + DF, jnp.int32)
                    vA0 = plsc.load_gather(tap_v, [row0, cA])
                    vB0 = plsc.load_gather(tap_v, [row0, cB])
                    vA1 = plsc.load_gather(tap_v, [row1, cA])
                    vB1 = plsc.load_gather(tap_v, [row1, cB])
                    acc = vA0 * w00 + vB0 * w01 + vA1 * w10 + vB1 * w11
                    plsc.store_scatter(out_v, [pt96 + (l * 24 + s * DF + c)], acc)
            return c2

        lax.fori_loop(0, NGRP, grp_comb, 0)

        pltpu.sync_copy(out_v, out.at[pl.ds(base * (NLVL * NPLANE * DF), CH * NLVL * NPLANE * DF)])
        return carry

    lax.fori_loop(0, NCHUNK, chunk_body, 0)


def _sc_lookup(x0, x1, x2, t0, t1, t2, t3):
    mesh = plsc.VectorSubcoreMesh(core_axis_name="c", subcore_axis_name="s")
    return pl.kernel(
        _tec_body,
        out_type=jax.ShapeDtypeStruct((NPTS * NLVL * NPLANE * DF,), jnp.float32),
        mesh=mesh,
        compiler_params=pltpu.CompilerParams(
            use_tc_tiling_on_sc=False, needs_layout_passes=False),
        scratch_types=[
            pltpu.VMEM((3 * CH,), jnp.float32),
            pltpu.VMEM((12, 2, CH), jnp.int32),
            pltpu.VMEM((12, 4, CH), jnp.float32),
            pltpu.VMEM((24 * CH, 16), jnp.float32),
            pltpu.VMEM((CH * NLVL * NPLANE * DF,), jnp.float32),
            pltpu.SemaphoreType.DMA,
        ],
    )(x0, x1, x2, t0, t1, t2, t3)


def _prep_table(plane, r):
    t = jnp.transpose(plane, (0, 2, 3, 1))
    pair = jnp.concatenate([t, jnp.roll(t, -1, axis=2)], axis=-1)
    return pair.reshape(NPLANE * r * r, 2 * DF)


def kernel(x, plane0, plane1, plane2, plane3):
    x0, x1, x2 = x[:, 0], x[:, 1], x[:, 2]
    tbls = [_prep_table(p, r)
            for p, r in zip((plane0, plane1, plane2, plane3), RES_LIST)]
    flat = _sc_lookup(x0, x1, x2, *tbls)
    return flat.reshape(NPTS, NLVL * NPLANE * DF)


# speedup vs baseline: 161.3964x; 1.5100x over previous
import jax
import jax.numpy as jnp
from jax import lax
from jax.experimental import pallas as pl
from jax.experimental.pallas import tpu as pltpu
from jax.experimental.pallas import tpu_sc as plsc

RES_LIST = (64, 128, 256, 512)
NLVL = 4
NPLANE = 3
DF = 8
NOUT = NLVL * NPLANE * DF
NPTS = 1048576
NCORES = 2
NSUB = 16
LANES = 16
NWORK = NCORES * NSUB
PPW = NPTS // NWORK
CH = 128
NCHUNK = PPW // CH
NGRP = CH // LANES
NLP = NLVL * NPLANE
PLANE_DXY = ((0, 2), (1, 0), (2, 1))


def _tec_body(x0, x1, x2, tbl0, tbl1, tbl2, tbl3, out,
              xs_v, idx_v, w_v, tap_v, out_v,
              xsem0, xsem1, gsem0, gsem1, osem0, osem1):
    tbls = (tbl0, tbl1, tbl2, tbl3)
    xcoords = (x0, x1, x2)
    xsems = (xsem0, xsem1)
    gsems = (gsem0, gsem1)
    osems = (osem0, osem1)
    wid = lax.axis_index("s") * NCORES + lax.axis_index("c")
    base0 = wid * PPW
    iota = lax.iota(jnp.int32, LANES)
    himask = jnp.full((LANES,), -65536, jnp.int32)

    def fire_xs(ci, b):
        base = base0 + ci * CH
        for d in range(3):
            pltpu.async_copy(xcoords[d].at[pl.ds(base, CH)],
                             xs_v.at[b, d], xsems[b])

    def wait_xs(b):
        for d in range(3):
            pltpu.make_async_copy(xcoords[d].at[pl.ds(0, CH)],
                                  xs_v.at[b, d], xsems[b]).wait()

    def compute_idx(b):
        def grp(g, c2):
            g0 = g * LANES
            for l in range(NLVL):
                r = RES_LIST[l]
                st = []
                for d in range(3):
                    xd = xs_v[b, d, pl.ds(g0, LANES)]
                    t = xd * float(r) - 0.5
                    i = (t + 1.0).astype(jnp.int32) - 1
                    w1 = t - i.astype(jnp.float32)
                    w0 = 1.0 - w1
                    lo = i < 0
                    hi = i > r - 2
                    ib = jnp.minimum(jnp.maximum(i, 0), r - 2)
                    wA = jnp.where(lo, w1, jnp.where(hi, 0.0, w0))
                    wB = jnp.where(hi, w0, jnp.where(lo, 0.0, w1))
                    st.append((ib, wA, wB))
                for s in range(NPLANE):
                    dx, dy = PLANE_DXY[s]
                    ibx, wxA, wxB = st[dx]
                    iby, wyA, wyB = st[dy]
                    lp = l * NPLANE + s
                    idx_v[b, lp, pl.ds(g0, LANES)] = iby * r + (ibx + s * r * r)
                    w_v[b, lp, 0, pl.ds(g0, LANES)] = wyA * wxA
                    w_v[b, lp, 1, pl.ds(g0, LANES)] = wyA * wxB
                    w_v[b, lp, 2, pl.ds(g0, LANES)] = wyB * wxA
                    w_v[b, lp, 3, pl.ds(g0, LANES)] = wyB * wxB
            return c2
        lax.fori_loop(0, NGRP, grp, 0)

    def fire_gathers(b):
        for l in range(NLVL):
            for s in range(NPLANE):
                lp = l * NPLANE + s
                pltpu.async_copy(tbls[l].at[idx_v.at[b, lp]],
                                 tap_v.at[b, pl.ds(lp * CH, CH), :],
                                 gsems[b])

    def wait_gathers(b):
        for l in range(NLVL):
            for s in range(NPLANE):
                lp = l * NPLANE + s
                pltpu.make_async_copy(tbls[l].at[idx_v.at[b, lp]],
                                      tap_v.at[b, pl.ds(lp * CH, CH), :],
                                      gsems[b]).wait()

    def combine(b):
        tap = tap_v.at[b]
        outb = out_v.at[b]

        def grp(g, c2):
            g0 = g * LANES
            pt = g0 + iota
            pt96 = pt * NOUT
            for lp in range(NLP):
                w00 = w_v[b, lp, 0, pl.ds(g0, LANES)]
                w01 = w_v[b, lp, 1, pl.ds(g0, LANES)]
                w10 = w_v[b, lp, 2, pl.ds(g0, LANES)]
                w11 = w_v[b, lp, 3, pl.ds(g0, LANES)]
                row = pt + lp * CH
                col0 = (lp // NPLANE) * 24 + (lp % NPLANE) * DF
                for k in range(4):
                    wa = plsc.load_gather(tap, [row, jnp.full((LANES,), k, jnp.int32)])
                    wb = plsc.load_gather(tap, [row, jnp.full((LANES,), k + 4, jnp.int32)])
                    wc = plsc.load_gather(tap, [row, jnp.full((LANES,), k + 8, jnp.int32)])
                    wd = plsc.load_gather(tap, [row, jnp.full((LANES,), k + 12, jnp.int32)])
                    alo = plsc.bitcast(wa << 16, jnp.float32)
                    blo = plsc.bitcast(wb << 16, jnp.float32)
                    clo = plsc.bitcast(wc << 16, jnp.float32)
                    dlo = plsc.bitcast(wd << 16, jnp.float32)
                    ahi = plsc.bitcast(wa & himask, jnp.float32)
                    bhi = plsc.bitcast(wb & himask, jnp.float32)
                    chi = plsc.bitcast(wc & himask, jnp.float32)
                    dhi = plsc.bitcast(wd & himask, jnp.float32)
                    even = alo * w00 + blo * w01 + clo * w10 + dlo * w11
                    odd = ahi * w00 + bhi * w01 + chi * w10 + dhi * w11
                    plsc.store_scatter(outb, [pt96 + (col0 + 2 * k)], even)
                    plsc.store_scatter(outb, [pt96 + (col0 + 2 * k + 1)], odd)
            return c2
        lax.fori_loop(0, NGRP, grp, 0)

    def fire_out(ci, b):
        base = base0 + ci * CH
        pltpu.async_copy(out_v.at[b], out.at[pl.ds(base * NOUT, CH * NOUT)],
                         osems[b])

    def wait_out(b):
        pltpu.make_async_copy(out_v.at[b], out.at[pl.ds(0, CH * NOUT)],
                              osems[b]).wait()

    fire_xs(0, 0)
    fire_xs(1, 1)
    wait_xs(0)
    compute_idx(0)
    fire_gathers(0)

    def pair_body(it, carry):
        for b in range(2):
            ci = it * 2 + b
            b1 = 1 - b
            @pl.when(ci + 1 < NCHUNK)
            def _():
                wait_xs(b1)
                compute_idx(b1)
                fire_gathers(b1)

            @pl.when(ci + 2 < NCHUNK)
            def _():
                fire_xs(ci + 2, b)

            @pl.when(ci >= 2)
            def _():
                wait_out(b)

            wait_gathers(b)
            combine(b)
            fire_out(ci, b)
        return carry

    lax.fori_loop(0, NCHUNK // 2, pair_body, 0)
    wait_out(0)
    wait_out(1)


def _sc_lookup(x0, x1, x2, t0, t1, t2, t3):
    mesh = plsc.VectorSubcoreMesh(core_axis_name="c", subcore_axis_name="s")
    return pl.kernel(
        _tec_body,
        out_type=jax.ShapeDtypeStruct((NPTS * NOUT,), jnp.float32),
        mesh=mesh,
        compiler_params=pltpu.CompilerParams(
            use_tc_tiling_on_sc=False, needs_layout_passes=False),
        scratch_types=[
            pltpu.VMEM((2, 3, CH), jnp.float32),
            pltpu.VMEM((2, NLP, CH), jnp.int32),
            pltpu.VMEM((2, NLP, 4, CH), jnp.float32),
            pltpu.VMEM((2, NLP * CH, 16), jnp.int32),
            pltpu.VMEM((2, CH * NOUT), jnp.float32),
            pltpu.SemaphoreType.DMA,
            pltpu.SemaphoreType.DMA,
            pltpu.SemaphoreType.DMA,
            pltpu.SemaphoreType.DMA,
            pltpu.SemaphoreType.DMA,
            pltpu.SemaphoreType.DMA,
        ],
    )(x0, x1, x2, t0, t1, t2, t3)


def _prep_table(plane, r):
    t = jnp.transpose(plane, (0, 2, 3, 1)).astype(jnp.bfloat16)
    tx = jnp.concatenate([t, jnp.roll(t, -1, axis=2)], axis=-1)
    q = jnp.concatenate([tx, jnp.roll(tx, -1, axis=1)], axis=-1)
    q = q.reshape(NPLANE * r * r, 16, 2)
    return lax.bitcast_convert_type(q, jnp.int32)


def kernel(x, plane0, plane1, plane2, plane3):
    x0, x1, x2 = x[:, 0], x[:, 1], x[:, 2]
    tbls = [_prep_table(p, r)
            for p, r in zip((plane0, plane1, plane2, plane3), RES_LIST)]
    flat = _sc_lookup(x0, x1, x2, *tbls)
    return flat.reshape(NPTS, NOUT)
